# Initial kernel scaffold; baseline (speedup 1.0000x reference)
import jax
import jax.numpy as jnp
from jax.experimental import pallas as pl


def kernel(x, adj, W_gcn, b_gcn, W_s, b_s, W_p, b_p, W_d1, b_d1, W_d2, b_d2, W_out, b_out):
    raise NotImplementedError("write your pallas kernel here")


# speedup vs baseline: 5.8375x; 5.8375x over previous
import functools

import jax
import jax.numpy as jnp
from jax import lax
from jax.experimental import pallas as pl
from jax.experimental.pallas import tpu as pltpu
from jax.experimental.pallas import tpu_sc as plsc

NUM_VIEW = 2
N = 10000
E = 320000
FT = 128
HID = 128
CDIM = 64
PDIM = 64

NTILE = 16
CHUNK = 128
IB = 8
NCH = 160
NIB = NCH // IB
EPT = NCH * CHUNK
EPAD = EPT * NTILE
TPB = 632
NP = TPB * NTILE

ROWB = 400
NROWB = N // ROWB


def _sc_agg_body(x2, srcp, dstp, z128, ones128, out, src_v, dst_v, rows_v,
                 acc_sh, sem):
    c = lax.axis_index("c")
    s = lax.axis_index("s")
    base = s * TPB

    def zero_acc(acc_sh):
        pltpu.sync_copy(z128, rows_v)
        for k in range(0, TPB, CHUNK):
            w = min(CHUNK, TPB - k)
            pltpu.sync_copy(rows_v.at[pl.ds(0, w)],
                            acc_sh.at[pl.ds(base + k, w)])

    def copy_out(acc_sh, phase):
        for k in range(0, TPB, CHUNK):
            w = min(CHUNK, TPB - k)
            pltpu.sync_copy(acc_sh.at[pl.ds(base + k, w)],
                            rows_v.at[pl.ds(0, w)])
            pltpu.sync_copy(rows_v.at[pl.ds(0, w)],
                            out.at[c, phase, pl.ds(base + k, w)])

    zero_acc(acc_sh)
    plsc.subcore_barrier()

    def step1(ib, carry):
        pltpu.sync_copy(srcp.at[c, s, pl.ds(ib * IB, IB)], src_v)
        pltpu.sync_copy(dstp.at[c, s, pl.ds(ib * IB, IB)], dst_v)
        for j in range(IB):
            pltpu.async_copy(x2.at[src_v.at[j]], rows_v, sem).wait()
            pltpu.sync_copy(rows_v, acc_sh.at[dst_v.at[j]], add=True)
        return carry

    lax.fori_loop(0, NIB, step1, 0)
    plsc.subcore_barrier()
    copy_out(acc_sh, 0)
    plsc.subcore_barrier()

    zero_acc(acc_sh)
    pltpu.sync_copy(ones128, rows_v)
    plsc.subcore_barrier()

    def step2(ib, carry):
        pltpu.sync_copy(dstp.at[c, s, pl.ds(ib * IB, IB)], dst_v)
        for j in range(IB):
            pltpu.sync_copy(rows_v, acc_sh.at[dst_v.at[j]], add=True)
        return carry

    lax.fori_loop(0, NIB, step2, 0)
    plsc.subcore_barrier()
    copy_out(acc_sh, 1)


@functools.cache
def _sc_agg():
    return pl.kernel(
        _sc_agg_body,
        out_type=jax.ShapeDtypeStruct((NUM_VIEW, 2, NP, FT), jnp.float32),
        mesh=plsc.VectorSubcoreMesh(core_axis_name="c", subcore_axis_name="s"),
        scratch_types=[
            pltpu.VMEM((IB, CHUNK), jnp.int32),
            pltpu.VMEM((IB, CHUNK), jnp.int32),
            pltpu.VMEM((CHUNK, FT), jnp.float32),
            pltpu.VMEM_SHARED((NP, FT), jnp.float32),
            pltpu.SemaphoreType.DMA,
        ],
    )


def _dense_body(agg, deg, wg, bg, ws, bs, wp, bp, wa, wb, bd1, wd2, bd2,
                wo, bo, com, priv, rec):
    f32 = jnp.float32
    dg = jnp.maximum(deg[0, :, 0:1], 1.0)
    a = agg[0] / dg
    h = jnp.maximum(jnp.dot(a, wg[0], preferred_element_type=f32) + bg[0], 0.0)
    cc = jnp.dot(h, ws[0], preferred_element_type=f32) + bs[0]
    pp = jnp.dot(h, wp[0], preferred_element_type=f32) + bp[0]
    d1 = jnp.maximum(jnp.dot(cc, wa[0], preferred_element_type=f32)
                     + jnp.dot(pp, wb[0], preferred_element_type=f32)
                     + bd1[0], 0.0)
    d2 = jnp.dot(d1, wd2[0], preferred_element_type=f32) + bd2[0]
    r = jnp.dot(jnp.maximum(d2, 0.0), wo[0], preferred_element_type=f32) + bo[0]
    com[0] = cc
    priv[0] = pp
    rec[0] = r


def _row_spec(d):
    return pl.BlockSpec((1, ROWB, d), lambda v, i: (v, i, 0))


def _w_spec(r, c):
    return pl.BlockSpec((1, r, c), lambda v, i: (v, 0, 0))


_dense = pl.pallas_call(
    _dense_body,
    grid=(NUM_VIEW, NROWB),
    in_specs=[
        _row_spec(FT),
        _row_spec(FT),
        _w_spec(FT, HID),
        _w_spec(1, HID),
        _w_spec(HID, CDIM),
        _w_spec(1, CDIM),
        _w_spec(HID, PDIM),
        _w_spec(1, PDIM),
        _w_spec(CDIM, HID),
        _w_spec(PDIM, HID),
        _w_spec(1, HID),
        _w_spec(HID, FT),
        _w_spec(1, FT),
        _w_spec(FT, FT),
        _w_spec(1, FT),
    ],
    out_specs=[_row_spec(CDIM), _row_spec(PDIM), _row_spec(FT)],
    out_shape=[
        jax.ShapeDtypeStruct((NUM_VIEW, N, CDIM), jnp.float32),
        jax.ShapeDtypeStruct((NUM_VIEW, N, PDIM), jnp.float32),
        jax.ShapeDtypeStruct((NUM_VIEW, N, FT), jnp.float32),
    ],
)


def kernel(x, adj, W_gcn, b_gcn, W_s, b_s, W_p, b_p, W_d1, b_d1, W_d2, b_d2,
           W_out, b_out):
    x2 = x.reshape(NUM_VIEW * N, FT)
    voff = (jnp.arange(NUM_VIEW, dtype=jnp.int32) * N)[:, None]
    pad = EPAD - E
    pad_src = jnp.broadcast_to((jnp.arange(pad, dtype=jnp.int32) * 31) % N,
                               (NUM_VIEW, pad)) + voff
    pad_dst = jnp.broadcast_to(N + (jnp.arange(pad, dtype=jnp.int32) % (NP - N)),
                               (NUM_VIEW, pad))
    src = jnp.concatenate([adj[:, 0, :] + voff, pad_src], axis=1)
    dst = jnp.concatenate([adj[:, 1, :], pad_dst], axis=1)
    srcp = src.reshape(NUM_VIEW, NTILE, NCH, CHUNK)
    dstp = dst.reshape(NUM_VIEW, NTILE, NCH, CHUNK)
    z128 = jnp.zeros((CHUNK, FT), jnp.float32)
    ones128 = jnp.ones((CHUNK, FT), jnp.float32)

    out = _sc_agg()(x2, srcp, dstp, z128, ones128)
    agg = out[:, 0, :N]
    deg = out[:, 1, :N]

    com, priv, rec = _dense(
        agg, deg, W_gcn, b_gcn.reshape(NUM_VIEW, 1, HID),
        W_s, b_s.reshape(NUM_VIEW, 1, CDIM),
        W_p, b_p.reshape(NUM_VIEW, 1, PDIM),
        W_d1[:, :CDIM], W_d1[:, CDIM:], b_d1.reshape(NUM_VIEW, 1, HID),
        W_d2, b_d2.reshape(NUM_VIEW, 1, FT),
        W_out, b_out.reshape(NUM_VIEW, 1, FT))
    return (com, priv, rec)


# speedup vs baseline: 6.0348x; 1.0338x over previous
import functools

import jax
import jax.numpy as jnp
from jax import lax
from jax.experimental import pallas as pl
from jax.experimental.pallas import tpu as pltpu
from jax.experimental.pallas import tpu_sc as plsc

NUM_VIEW = 2
N = 10000
E = 320000
FT = 128
HID = 128
CDIM = 64
PDIM = 64

NTILE = 16
CHUNK = 80
IB = 10
NCH = 250
NIB = NCH // IB
EPT = NCH * CHUNK
EPAD = EPT * NTILE
TPB = 632
NP = TPB * NTILE

ROWB = 400
NROWB = N // ROWB


def _sc_agg_body(x2, srcp, dstp, z128, ones128, out, src_v, dst_v, rows_a,
                 rows_b, acc_sh, sem_a, sem_b):
    c = lax.axis_index("c")
    s = lax.axis_index("s")
    base = s * TPB
    rows = (rows_a, rows_b)
    sems = (sem_a, sem_b)

    def zero_acc():
        pltpu.sync_copy(z128, rows_a)
        for k in range(0, TPB, CHUNK):
            w = min(CHUNK, TPB - k)
            pltpu.sync_copy(rows_a.at[pl.ds(0, w)],
                            acc_sh.at[pl.ds(base + k, w)])

    def copy_out(phase):
        for k in range(0, TPB, CHUNK):
            w = min(CHUNK, TPB - k)
            pltpu.sync_copy(acc_sh.at[pl.ds(base + k, w)],
                            rows_a.at[pl.ds(0, w)])
            pltpu.sync_copy(rows_a.at[pl.ds(0, w)],
                            out.at[c, phase, pl.ds(base + k, w)])

    zero_acc()
    plsc.subcore_barrier()

    def step1(ib, carry):
        pltpu.sync_copy(srcp.at[c, s, ib], src_v)
        pltpu.sync_copy(dstp.at[c, s, ib], dst_v)
        pend = pltpu.async_copy(x2.at[src_v.at[0]], rows[0], sems[0])
        for j in range(IB):
            pend.wait()
            if j + 1 < IB:
                pend = pltpu.async_copy(x2.at[src_v.at[j + 1]],
                                        rows[(j + 1) % 2], sems[(j + 1) % 2])
            pltpu.sync_copy(rows[j % 2], acc_sh.at[dst_v.at[j]], add=True)
        return carry

    lax.fori_loop(0, NIB, step1, 0)
    plsc.subcore_barrier()
    copy_out(0)
    plsc.subcore_barrier()

    zero_acc()
    pltpu.sync_copy(ones128, rows_b)
    plsc.subcore_barrier()

    def step2(ib, carry):
        pltpu.sync_copy(dstp.at[c, s, ib], dst_v)
        for j in range(IB):
            pltpu.sync_copy(rows_b, acc_sh.at[dst_v.at[j]], add=True)
        return carry

    lax.fori_loop(0, NIB, step2, 0)
    plsc.subcore_barrier()
    copy_out(1)


@functools.cache
def _sc_agg():
    return pl.kernel(
        _sc_agg_body,
        out_type=jax.ShapeDtypeStruct((NUM_VIEW, 2, NP, FT), jnp.float32),
        mesh=plsc.VectorSubcoreMesh(core_axis_name="c", subcore_axis_name="s"),
        scratch_types=[
            pltpu.VMEM((IB, CHUNK), jnp.int32),
            pltpu.VMEM((IB, CHUNK), jnp.int32),
            pltpu.VMEM((CHUNK, FT), jnp.float32),
            pltpu.VMEM((CHUNK, FT), jnp.float32),
            pltpu.VMEM_SHARED((NP, FT), jnp.float32),
            pltpu.SemaphoreType.DMA,
            pltpu.SemaphoreType.DMA,
        ],
    )


def _dense_body(agg, deg, wg, bg, ws, bs, wp, bp, wa, wb, bd1, wd2, bd2,
                wo, bo, com, priv, rec):
    f32 = jnp.float32
    dg = jnp.maximum(deg[0, :, 0:1], 1.0)
    a = agg[0] / dg
    h = jnp.maximum(jnp.dot(a, wg[0], preferred_element_type=f32) + bg[0], 0.0)
    cc = jnp.dot(h, ws[0], preferred_element_type=f32) + bs[0]
    pp = jnp.dot(h, wp[0], preferred_element_type=f32) + bp[0]
    d1 = jnp.maximum(jnp.dot(cc, wa[0], preferred_element_type=f32)
                     + jnp.dot(pp, wb[0], preferred_element_type=f32)
                     + bd1[0], 0.0)
    d2 = jnp.dot(d1, wd2[0], preferred_element_type=f32) + bd2[0]
    r = jnp.dot(jnp.maximum(d2, 0.0), wo[0], preferred_element_type=f32) + bo[0]
    com[0] = cc
    priv[0] = pp
    rec[0] = r


def _row_spec(d):
    return pl.BlockSpec((1, ROWB, d), lambda v, i: (v, i, 0))


def _w_spec(r, c):
    return pl.BlockSpec((1, r, c), lambda v, i: (v, 0, 0))


_dense = pl.pallas_call(
    _dense_body,
    grid=(NUM_VIEW, NROWB),
    in_specs=[
        _row_spec(FT),
        _row_spec(FT),
        _w_spec(FT, HID),
        _w_spec(1, HID),
        _w_spec(HID, CDIM),
        _w_spec(1, CDIM),
        _w_spec(HID, PDIM),
        _w_spec(1, PDIM),
        _w_spec(CDIM, HID),
        _w_spec(PDIM, HID),
        _w_spec(1, HID),
        _w_spec(HID, FT),
        _w_spec(1, FT),
        _w_spec(FT, FT),
        _w_spec(1, FT),
    ],
    out_specs=[_row_spec(CDIM), _row_spec(PDIM), _row_spec(FT)],
    out_shape=[
        jax.ShapeDtypeStruct((NUM_VIEW, N, CDIM), jnp.float32),
        jax.ShapeDtypeStruct((NUM_VIEW, N, PDIM), jnp.float32),
        jax.ShapeDtypeStruct((NUM_VIEW, N, FT), jnp.float32),
    ],
)


def kernel(x, adj, W_gcn, b_gcn, W_s, b_s, W_p, b_p, W_d1, b_d1, W_d2, b_d2,
           W_out, b_out):
    x2 = x.reshape(NUM_VIEW * N, FT)
    voff = (jnp.arange(NUM_VIEW, dtype=jnp.int32) * N)[:, None]
    pad = EPAD - E
    src = adj[:, 0, :] + voff
    dst = adj[:, 1, :]
    if pad:
        pad_src = jnp.broadcast_to((jnp.arange(pad, dtype=jnp.int32) * 31) % N,
                                   (NUM_VIEW, pad)) + voff
        pad_dst = jnp.broadcast_to(
            N + (jnp.arange(pad, dtype=jnp.int32) % (NP - N)), (NUM_VIEW, pad))
        src = jnp.concatenate([src, pad_src], axis=1)
        dst = jnp.concatenate([dst, pad_dst], axis=1)
    srcp = src.reshape(NUM_VIEW, NTILE, NIB, IB, CHUNK)
    dstp = dst.reshape(NUM_VIEW, NTILE, NIB, IB, CHUNK)
    z128 = jnp.zeros((CHUNK, FT), jnp.float32)
    ones128 = jnp.ones((CHUNK, FT), jnp.float32)

    out = _sc_agg()(x2, srcp, dstp, z128, ones128)
    agg = out[:, 0, :N]
    deg = out[:, 1, :N]

    com, priv, rec = _dense(
        agg, deg, W_gcn, b_gcn.reshape(NUM_VIEW, 1, HID),
        W_s, b_s.reshape(NUM_VIEW, 1, CDIM),
        W_p, b_p.reshape(NUM_VIEW, 1, PDIM),
        W_d1[:, :CDIM], W_d1[:, CDIM:], b_d1.reshape(NUM_VIEW, 1, HID),
        W_d2, b_d2.reshape(NUM_VIEW, 1, FT),
        W_out, b_out.reshape(NUM_VIEW, 1, FT))
    return (com, priv, rec)


# speedup vs baseline: 6.0864x; 1.0086x over previous
import functools

import jax
import jax.numpy as jnp
from jax import lax
from jax.experimental import pallas as pl
from jax.experimental.pallas import tpu as pltpu
from jax.experimental.pallas import tpu_sc as plsc

NUM_VIEW = 2
N = 10000
E = 320000
FT = 128
HID = 128
CDIM = 64
PDIM = 64

NTILE = 16
CHUNK = 80
IB = 10
NCH = 250
NIB = NCH // IB
EPT = NCH * CHUNK
EPAD = EPT * NTILE
TPB = 632
NP = TPB * NTILE

ROWB = 400
NROWB = N // ROWB


def _sc_agg_body(x2, srcp, dstp, z128, ones128, out, src_v, dst_v, rows_a,
                 rows_b, acc_sh, sem_a, sem_b, ssem_a, ssem_b):
    c = lax.axis_index("c")
    s = lax.axis_index("s")
    base = s * TPB
    rows = (rows_a, rows_b)
    sems = (sem_a, sem_b)
    ssems = (ssem_a, ssem_b)

    def zero_acc():
        pltpu.sync_copy(z128, rows_a)
        for k in range(0, TPB, CHUNK):
            w = min(CHUNK, TPB - k)
            pltpu.sync_copy(rows_a.at[pl.ds(0, w)],
                            acc_sh.at[pl.ds(base + k, w)])

    def copy_out(phase):
        for k in range(0, TPB, CHUNK):
            w = min(CHUNK, TPB - k)
            pltpu.sync_copy(acc_sh.at[pl.ds(base + k, w)],
                            rows_a.at[pl.ds(0, w)])
            pltpu.sync_copy(rows_a.at[pl.ds(0, w)],
                            out.at[c, phase, pl.ds(base + k, w)])

    zero_acc()
    plsc.subcore_barrier()

    def step1(ib, carry):
        pltpu.sync_copy(srcp.at[c, s, ib], src_v)
        pltpu.sync_copy(dstp.at[c, s, ib], dst_v)
        pend = pltpu.async_copy(x2.at[src_v.at[0]], rows[0], sems[0])
        scs = [None, None]
        for j in range(IB):
            pend.wait()
            sc = pltpu.async_copy(rows[j % 2], acc_sh.at[dst_v.at[j]],
                                  ssems[j % 2], add=True)
            if j + 1 < IB:
                if scs[(j + 1) % 2] is not None:
                    scs[(j + 1) % 2].wait()
                pend = pltpu.async_copy(x2.at[src_v.at[j + 1]],
                                        rows[(j + 1) % 2], sems[(j + 1) % 2])
            scs[j % 2] = sc
        scs[(IB - 1) % 2].wait()
        scs[IB % 2].wait()
        return carry

    lax.fori_loop(0, NIB, step1, 0)
    plsc.subcore_barrier()
    copy_out(0)
    plsc.subcore_barrier()

    zero_acc()
    pltpu.sync_copy(ones128, rows_b)
    plsc.subcore_barrier()

    def step2(ib, carry):
        pltpu.sync_copy(dstp.at[c, s, ib], dst_v)
        descs = [pltpu.async_copy(rows_b, acc_sh.at[dst_v.at[j]], ssem_a,
                                  add=True) for j in range(IB)]
        for d in descs:
            d.wait()
        return carry

    lax.fori_loop(0, NIB, step2, 0)
    plsc.subcore_barrier()
    copy_out(1)


@functools.cache
def _sc_agg():
    return pl.kernel(
        _sc_agg_body,
        out_type=jax.ShapeDtypeStruct((NUM_VIEW, 2, NP, FT), jnp.float32),
        mesh=plsc.VectorSubcoreMesh(core_axis_name="c", subcore_axis_name="s"),
        scratch_types=[
            pltpu.VMEM((IB, CHUNK), jnp.int32),
            pltpu.VMEM((IB, CHUNK), jnp.int32),
            pltpu.VMEM((CHUNK, FT), jnp.float32),
            pltpu.VMEM((CHUNK, FT), jnp.float32),
            pltpu.VMEM_SHARED((NP, FT), jnp.float32),
            pltpu.SemaphoreType.DMA,
            pltpu.SemaphoreType.DMA,
            pltpu.SemaphoreType.DMA,
            pltpu.SemaphoreType.DMA,
        ],
    )


def _dense_body(agg, deg, wg, bg, ws, bs, wp, bp, wa, wb, bd1, wd2, bd2,
                wo, bo, com, priv, rec):
    f32 = jnp.float32
    dg = jnp.maximum(deg[0, :, 0:1], 1.0)
    a = agg[0] / dg
    h = jnp.maximum(jnp.dot(a, wg[0], preferred_element_type=f32) + bg[0], 0.0)
    cc = jnp.dot(h, ws[0], preferred_element_type=f32) + bs[0]
    pp = jnp.dot(h, wp[0], preferred_element_type=f32) + bp[0]
    d1 = jnp.maximum(jnp.dot(cc, wa[0], preferred_element_type=f32)
                     + jnp.dot(pp, wb[0], preferred_element_type=f32)
                     + bd1[0], 0.0)
    d2 = jnp.dot(d1, wd2[0], preferred_element_type=f32) + bd2[0]
    r = jnp.dot(jnp.maximum(d2, 0.0), wo[0], preferred_element_type=f32) + bo[0]
    com[0] = cc
    priv[0] = pp
    rec[0] = r


def _row_spec(d):
    return pl.BlockSpec((1, ROWB, d), lambda v, i: (v, i, 0))


def _w_spec(r, c):
    return pl.BlockSpec((1, r, c), lambda v, i: (v, 0, 0))


_dense = pl.pallas_call(
    _dense_body,
    grid=(NUM_VIEW, NROWB),
    in_specs=[
        _row_spec(FT),
        _row_spec(FT),
        _w_spec(FT, HID),
        _w_spec(1, HID),
        _w_spec(HID, CDIM),
        _w_spec(1, CDIM),
        _w_spec(HID, PDIM),
        _w_spec(1, PDIM),
        _w_spec(CDIM, HID),
        _w_spec(PDIM, HID),
        _w_spec(1, HID),
        _w_spec(HID, FT),
        _w_spec(1, FT),
        _w_spec(FT, FT),
        _w_spec(1, FT),
    ],
    out_specs=[_row_spec(CDIM), _row_spec(PDIM), _row_spec(FT)],
    out_shape=[
        jax.ShapeDtypeStruct((NUM_VIEW, N, CDIM), jnp.float32),
        jax.ShapeDtypeStruct((NUM_VIEW, N, PDIM), jnp.float32),
        jax.ShapeDtypeStruct((NUM_VIEW, N, FT), jnp.float32),
    ],
)


def kernel(x, adj, W_gcn, b_gcn, W_s, b_s, W_p, b_p, W_d1, b_d1, W_d2, b_d2,
           W_out, b_out):
    x2 = x.reshape(NUM_VIEW * N, FT)
    voff = (jnp.arange(NUM_VIEW, dtype=jnp.int32) * N)[:, None]
    pad = EPAD - E
    src = adj[:, 0, :] + voff
    dst = adj[:, 1, :]
    if pad:
        pad_src = jnp.broadcast_to((jnp.arange(pad, dtype=jnp.int32) * 31) % N,
                                   (NUM_VIEW, pad)) + voff
        pad_dst = jnp.broadcast_to(
            N + (jnp.arange(pad, dtype=jnp.int32) % (NP - N)), (NUM_VIEW, pad))
        src = jnp.concatenate([src, pad_src], axis=1)
        dst = jnp.concatenate([dst, pad_dst], axis=1)
    srcp = src.reshape(NUM_VIEW, NTILE, NIB, IB, CHUNK)
    dstp = dst.reshape(NUM_VIEW, NTILE, NIB, IB, CHUNK)
    z128 = jnp.zeros((CHUNK, FT), jnp.float32)
    ones128 = jnp.ones((CHUNK, FT), jnp.float32)

    out = _sc_agg()(x2, srcp, dstp, z128, ones128)
    agg = out[:, 0, :N]
    deg = out[:, 1, :N]

    com, priv, rec = _dense(
        agg, deg, W_gcn, b_gcn.reshape(NUM_VIEW, 1, HID),
        W_s, b_s.reshape(NUM_VIEW, 1, CDIM),
        W_p, b_p.reshape(NUM_VIEW, 1, PDIM),
        W_d1[:, :CDIM], W_d1[:, CDIM:], b_d1.reshape(NUM_VIEW, 1, HID),
        W_d2, b_d2.reshape(NUM_VIEW, 1, FT),
        W_out, b_out.reshape(NUM_VIEW, 1, FT))
    return (com, priv, rec)


# speedup vs baseline: 6.2006x; 1.0188x over previous
import functools

import jax
import jax.numpy as jnp
from jax import lax
from jax.experimental import pallas as pl
from jax.experimental.pallas import tpu as pltpu
from jax.experimental.pallas import tpu_sc as plsc

NUM_VIEW = 2
N = 10000
E = 320000
FT = 128
HID = 128
CDIM = 64
PDIM = 64

NTILE = 16
CHUNK = 80
IB = 10
NCH = 250
NIB = NCH // IB
EPT = NCH * CHUNK
EPAD = EPT * NTILE
TPB = 632
NP = TPB * NTILE

ROWB = 400
NROWB = N // ROWB


def _sc_agg_body(x2, srcp, dstp, z128, ones128, out, src_v, dst_v, rows_a,
                 rows_b, acc_sh, sem_a, sem_b, ssem_a, ssem_b):
    c = lax.axis_index("c")
    s = lax.axis_index("s")
    base = s * TPB
    rows = (rows_a, rows_b)
    sems = (sem_a, sem_b)
    ssems = (ssem_a, ssem_b)

    def zero_acc():
        pltpu.sync_copy(z128, rows_a)
        for k in range(0, TPB, CHUNK):
            w = min(CHUNK, TPB - k)
            pltpu.sync_copy(rows_a.at[pl.ds(0, w)],
                            acc_sh.at[pl.ds(base + k, w)])

    def copy_out(phase):
        for k in range(0, TPB, CHUNK):
            w = min(CHUNK, TPB - k)
            pltpu.sync_copy(acc_sh.at[pl.ds(base + k, w)],
                            rows_a.at[pl.ds(0, w)])
            pltpu.sync_copy(rows_a.at[pl.ds(0, w)],
                            out.at[c, phase, pl.ds(base + k, w)])

    zero_acc()
    plsc.subcore_barrier()

    def step1(ib, carry):
        pltpu.sync_copy(srcp.at[c, s, ib], src_v)
        pltpu.sync_copy(dstp.at[c, s, ib], dst_v)
        pend = pltpu.async_copy(x2.at[src_v.at[0]], rows[0], sems[0])
        scs = [None, None]
        for j in range(IB):
            pend.wait()
            sc = pltpu.async_copy(rows[j % 2], acc_sh.at[dst_v.at[j]],
                                  ssems[j % 2], add=True)
            if j + 1 < IB:
                if scs[(j + 1) % 2] is not None:
                    scs[(j + 1) % 2].wait()
                pend = pltpu.async_copy(x2.at[src_v.at[j + 1]],
                                        rows[(j + 1) % 2], sems[(j + 1) % 2])
            scs[j % 2] = sc
        scs[(IB - 1) % 2].wait()
        scs[IB % 2].wait()
        return carry

    lax.fori_loop(0, NIB, step1, 0)
    plsc.subcore_barrier()
    copy_out(0)
    plsc.subcore_barrier()

    zero_acc()
    pltpu.sync_copy(ones128, rows_b)
    plsc.subcore_barrier()

    def step2(ib, carry):
        pltpu.sync_copy(dstp.at[c, s, ib], dst_v)
        descs = [pltpu.async_copy(rows_b, acc_sh.at[dst_v.at[j]], ssem_a,
                                  add=True) for j in range(IB)]
        for d in descs:
            d.wait()
        return carry

    lax.fori_loop(0, NIB, step2, 0)
    plsc.subcore_barrier()
    copy_out(1)


@functools.cache
def _sc_agg():
    return pl.kernel(
        _sc_agg_body,
        out_type=jax.ShapeDtypeStruct((NUM_VIEW, 2, NP, FT), jnp.float32),
        mesh=plsc.VectorSubcoreMesh(core_axis_name="c", subcore_axis_name="s"),
        scratch_types=[
            pltpu.VMEM((IB, CHUNK), jnp.int32),
            pltpu.VMEM((IB, CHUNK), jnp.int32),
            pltpu.VMEM((CHUNK, FT), jnp.float32),
            pltpu.VMEM((CHUNK, FT), jnp.float32),
            pltpu.VMEM_SHARED((NP, FT), jnp.float32),
            pltpu.SemaphoreType.DMA,
            pltpu.SemaphoreType.DMA,
            pltpu.SemaphoreType.DMA,
            pltpu.SemaphoreType.DMA,
        ],
    )


def _dense_body(acc, wg, bg, ws, bs, wp, bp, wa, wb, bd1, wd2, bd2,
                wo, bo, com, priv, rec):
    f32 = jnp.float32
    dg = jnp.maximum(acc[0, 1, :, 0:1], 1.0)
    a = acc[0, 0] / dg
    h = jnp.maximum(jnp.dot(a, wg[0], preferred_element_type=f32) + bg[0], 0.0)
    cc = jnp.dot(h, ws[0], preferred_element_type=f32) + bs[0]
    pp = jnp.dot(h, wp[0], preferred_element_type=f32) + bp[0]
    d1 = jnp.maximum(jnp.dot(cc, wa[0], preferred_element_type=f32)
                     + jnp.dot(pp, wb[0], preferred_element_type=f32)
                     + bd1[0], 0.0)
    d2 = jnp.dot(d1, wd2[0], preferred_element_type=f32) + bd2[0]
    r = jnp.dot(jnp.maximum(d2, 0.0), wo[0], preferred_element_type=f32) + bo[0]
    com[0] = cc
    priv[0] = pp
    rec[0] = r


def _row_spec(d):
    return pl.BlockSpec((1, ROWB, d), lambda v, i: (v, i, 0))


def _w_spec(r, c):
    return pl.BlockSpec((1, r, c), lambda v, i: (v, 0, 0))


_dense = pl.pallas_call(
    _dense_body,
    grid=(NUM_VIEW, NROWB),
    in_specs=[
        pl.BlockSpec((1, 2, ROWB, FT), lambda v, i: (v, 0, i, 0)),
        _w_spec(FT, HID),
        _w_spec(1, HID),
        _w_spec(HID, CDIM),
        _w_spec(1, CDIM),
        _w_spec(HID, PDIM),
        _w_spec(1, PDIM),
        _w_spec(CDIM, HID),
        _w_spec(PDIM, HID),
        _w_spec(1, HID),
        _w_spec(HID, FT),
        _w_spec(1, FT),
        _w_spec(FT, FT),
        _w_spec(1, FT),
    ],
    out_specs=[_row_spec(CDIM), _row_spec(PDIM), _row_spec(FT)],
    out_shape=[
        jax.ShapeDtypeStruct((NUM_VIEW, N, CDIM), jnp.float32),
        jax.ShapeDtypeStruct((NUM_VIEW, N, PDIM), jnp.float32),
        jax.ShapeDtypeStruct((NUM_VIEW, N, FT), jnp.float32),
    ],
)


def kernel(x, adj, W_gcn, b_gcn, W_s, b_s, W_p, b_p, W_d1, b_d1, W_d2, b_d2,
           W_out, b_out):
    x2 = x.reshape(NUM_VIEW * N, FT)
    voff = (jnp.arange(NUM_VIEW, dtype=jnp.int32) * N)[:, None]
    pad = EPAD - E
    src = adj[:, 0, :] + voff
    dst = adj[:, 1, :]
    if pad:
        pad_src = jnp.broadcast_to((jnp.arange(pad, dtype=jnp.int32) * 31) % N,
                                   (NUM_VIEW, pad)) + voff
        pad_dst = jnp.broadcast_to(
            N + (jnp.arange(pad, dtype=jnp.int32) % (NP - N)), (NUM_VIEW, pad))
        src = jnp.concatenate([src, pad_src], axis=1)
        dst = jnp.concatenate([dst, pad_dst], axis=1)
    srcp = src.reshape(NUM_VIEW, NTILE, NIB, IB, CHUNK)
    dstp = dst.reshape(NUM_VIEW, NTILE, NIB, IB, CHUNK)
    z128 = jnp.zeros((CHUNK, FT), jnp.float32)
    ones128 = jnp.ones((CHUNK, FT), jnp.float32)

    out = _sc_agg()(x2, srcp, dstp, z128, ones128)

    com, priv, rec = _dense(
        out, W_gcn, b_gcn.reshape(NUM_VIEW, 1, HID),
        W_s, b_s.reshape(NUM_VIEW, 1, CDIM),
        W_p, b_p.reshape(NUM_VIEW, 1, PDIM),
        W_d1[:, :CDIM], W_d1[:, CDIM:], b_d1.reshape(NUM_VIEW, 1, HID),
        W_d2, b_d2.reshape(NUM_VIEW, 1, FT),
        W_out, b_out.reshape(NUM_VIEW, 1, FT))
    return (com, priv, rec)


# speedup vs baseline: 6.2603x; 1.0096x over previous
import functools

import jax
import jax.numpy as jnp
from jax import lax
from jax.experimental import pallas as pl
from jax.experimental.pallas import tpu as pltpu
from jax.experimental.pallas import tpu_sc as plsc

NUM_VIEW = 2
N = 10000
E = 320000
FT = 128
HID = 128
CDIM = 64
PDIM = 64

NTILE = 16
CHUNK = 80
IB = 10
NCH = 250
NIB = NCH // IB
EPT = NCH * CHUNK
EPAD = EPT * NTILE
TPB = 632
NP = TPB * NTILE

ROWB = 400
NROWB = N // ROWB


def _sc_agg_body(x2, srcp, dstp, z128, ones128, out, src_v, dst_v, rows_a,
                 rows_b, acc_sh, sem_a, sem_b, ssem_a, ssem_b):
    c = lax.axis_index("c")
    s = lax.axis_index("s")
    base = s * TPB
    rows = (rows_a, rows_b)
    sems = (sem_a, sem_b)
    ssems = (ssem_a, ssem_b)

    def zero_acc():
        pltpu.sync_copy(z128, rows_a)
        for k in range(0, TPB, CHUNK):
            w = min(CHUNK, TPB - k)
            pltpu.sync_copy(rows_a.at[pl.ds(0, w)],
                            acc_sh.at[pl.ds(base + k, w)])

    def copy_out(phase):
        for k in range(0, TPB, CHUNK):
            w = min(CHUNK, TPB - k)
            pltpu.sync_copy(acc_sh.at[pl.ds(base + k, w)],
                            rows_a.at[pl.ds(0, w)])
            pltpu.sync_copy(rows_a.at[pl.ds(0, w)],
                            out.at[c, phase, pl.ds(base + k, w)])

    zero_acc()
    plsc.subcore_barrier()

    def step1(ib, carry):
        pltpu.sync_copy(srcp.at[c, s, ib], src_v)
        pltpu.sync_copy(dstp.at[c, s, ib], dst_v)
        pend = pltpu.async_copy(x2.at[src_v.at[0]], rows[0], sems[0])
        scs = [None, None]
        for j in range(IB):
            pend.wait()
            sc = pltpu.async_copy(rows[j % 2], acc_sh.at[dst_v.at[j]],
                                  ssems[j % 2], add=True)
            if j + 1 < IB:
                if scs[(j + 1) % 2] is not None:
                    scs[(j + 1) % 2].wait()
                pend = pltpu.async_copy(x2.at[src_v.at[j + 1]],
                                        rows[(j + 1) % 2], sems[(j + 1) % 2])
            scs[j % 2] = sc
        scs[(IB - 1) % 2].wait()
        scs[IB % 2].wait()
        return carry

    lax.fori_loop(0, NIB, step1, 0)
    plsc.subcore_barrier()
    copy_out(0)
    pltpu.sync_copy(ones128, rows_b)
    plsc.subcore_barrier()


    def step2(ib, carry):
        pltpu.sync_copy(dstp.at[c, s, ib], dst_v)
        descs = [pltpu.async_copy(rows_b, acc_sh.at[dst_v.at[j]], ssem_a,
                                  add=True) for j in range(IB)]
        for d in descs:
            d.wait()
        return carry

    lax.fori_loop(0, NIB, step2, 0)
    plsc.subcore_barrier()
    copy_out(1)


@functools.cache
def _sc_agg():
    return pl.kernel(
        _sc_agg_body,
        out_type=jax.ShapeDtypeStruct((NUM_VIEW, 2, NP, FT), jnp.float32),
        mesh=plsc.VectorSubcoreMesh(core_axis_name="c", subcore_axis_name="s"),
        scratch_types=[
            pltpu.VMEM((IB, CHUNK), jnp.int32),
            pltpu.VMEM((IB, CHUNK), jnp.int32),
            pltpu.VMEM((CHUNK, FT), jnp.float32),
            pltpu.VMEM((CHUNK, FT), jnp.float32),
            pltpu.VMEM_SHARED((NP, FT), jnp.float32),
            pltpu.SemaphoreType.DMA,
            pltpu.SemaphoreType.DMA,
            pltpu.SemaphoreType.DMA,
            pltpu.SemaphoreType.DMA,
        ],
    )


def _dense_body(acc, wg, bg, ws, bs, wp, bp, wa, wb, bd1, wd2, bd2,
                wo, bo, com, priv, rec):
    f32 = jnp.float32
    dg = jnp.maximum(acc[0, 1, :, 0:1] - acc[0, 0, :, 0:1], 1.0)
    a = acc[0, 0] / dg
    h = jnp.maximum(jnp.dot(a, wg[0], preferred_element_type=f32) + bg[0], 0.0)
    cc = jnp.dot(h, ws[0], preferred_element_type=f32) + bs[0]
    pp = jnp.dot(h, wp[0], preferred_element_type=f32) + bp[0]
    d1 = jnp.maximum(jnp.dot(cc, wa[0], preferred_element_type=f32)
                     + jnp.dot(pp, wb[0], preferred_element_type=f32)
                     + bd1[0], 0.0)
    d2 = jnp.dot(d1, wd2[0], preferred_element_type=f32) + bd2[0]
    r = jnp.dot(jnp.maximum(d2, 0.0), wo[0], preferred_element_type=f32) + bo[0]
    com[0] = cc
    priv[0] = pp
    rec[0] = r


def _row_spec(d):
    return pl.BlockSpec((1, ROWB, d), lambda v, i: (v, i, 0))


def _w_spec(r, c):
    return pl.BlockSpec((1, r, c), lambda v, i: (v, 0, 0))


_dense = pl.pallas_call(
    _dense_body,
    grid=(NUM_VIEW, NROWB),
    in_specs=[
        pl.BlockSpec((1, 2, ROWB, FT), lambda v, i: (v, 0, i, 0)),
        _w_spec(FT, HID),
        _w_spec(1, HID),
        _w_spec(HID, CDIM),
        _w_spec(1, CDIM),
        _w_spec(HID, PDIM),
        _w_spec(1, PDIM),
        _w_spec(CDIM, HID),
        _w_spec(PDIM, HID),
        _w_spec(1, HID),
        _w_spec(HID, FT),
        _w_spec(1, FT),
        _w_spec(FT, FT),
        _w_spec(1, FT),
    ],
    out_specs=[_row_spec(CDIM), _row_spec(PDIM), _row_spec(FT)],
    out_shape=[
        jax.ShapeDtypeStruct((NUM_VIEW, N, CDIM), jnp.float32),
        jax.ShapeDtypeStruct((NUM_VIEW, N, PDIM), jnp.float32),
        jax.ShapeDtypeStruct((NUM_VIEW, N, FT), jnp.float32),
    ],
)


def kernel(x, adj, W_gcn, b_gcn, W_s, b_s, W_p, b_p, W_d1, b_d1, W_d2, b_d2,
           W_out, b_out):
    x2 = x.reshape(NUM_VIEW * N, FT)
    voff = (jnp.arange(NUM_VIEW, dtype=jnp.int32) * N)[:, None]
    pad = EPAD - E
    src = adj[:, 0, :] + voff
    dst = adj[:, 1, :]
    if pad:
        pad_src = jnp.broadcast_to((jnp.arange(pad, dtype=jnp.int32) * 31) % N,
                                   (NUM_VIEW, pad)) + voff
        pad_dst = jnp.broadcast_to(
            N + (jnp.arange(pad, dtype=jnp.int32) % (NP - N)), (NUM_VIEW, pad))
        src = jnp.concatenate([src, pad_src], axis=1)
        dst = jnp.concatenate([dst, pad_dst], axis=1)
    srcp = src.reshape(NUM_VIEW, NTILE, NIB, IB, CHUNK)
    dstp = dst.reshape(NUM_VIEW, NTILE, NIB, IB, CHUNK)
    z128 = jnp.zeros((CHUNK, FT), jnp.float32)
    ones128 = jnp.ones((CHUNK, FT), jnp.float32)

    out = _sc_agg()(x2, srcp, dstp, z128, ones128)

    com, priv, rec = _dense(
        out, W_gcn, b_gcn.reshape(NUM_VIEW, 1, HID),
        W_s, b_s.reshape(NUM_VIEW, 1, CDIM),
        W_p, b_p.reshape(NUM_VIEW, 1, PDIM),
        W_d1[:, :CDIM], W_d1[:, CDIM:], b_d1.reshape(NUM_VIEW, 1, HID),
        W_d2, b_d2.reshape(NUM_VIEW, 1, FT),
        W_out, b_out.reshape(NUM_VIEW, 1, FT))
    return (com, priv, rec)


# speedup vs baseline: 6.7219x; 1.0737x over previous
import functools

import jax
import jax.numpy as jnp
from jax import lax
from jax.experimental import pallas as pl
from jax.experimental.pallas import tpu as pltpu
from jax.experimental.pallas import tpu_sc as plsc

NUM_VIEW = 2
N = 10000
E = 320000
FT = 128
HID = 128
CDIM = 64
PDIM = 64

NTILE = 16
CHUNK = 100
IB = 10
NCH = 200
NIB = NCH // IB
EPT = NCH * CHUNK
EPAD = EPT * NTILE
TPB = 632
NP = TPB * NTILE
SCB = 96

ROWB = 400
NROWB = N // ROWB


def _sc_agg_body(x2, srcp, dstp, z128, ones128, out, src_v, dst_v, rows_a,
                 rows_b, acc_sh, sem_a, sem_b, ssem_a, ssem_b):
    c = lax.axis_index("c")
    s = lax.axis_index("s")
    base = s * TPB
    rows = (rows_a, rows_b)
    sems = (sem_a, sem_b)
    ssems = (ssem_a, ssem_b)

    def zero_acc():
        pltpu.sync_copy(z128, rows_a)
        for k in range(0, TPB, SCB):
            w = min(SCB, TPB - k)
            pltpu.sync_copy(rows_a.at[pl.ds(0, w)],
                            acc_sh.at[pl.ds(base + k, w)])

    def copy_out(phase):
        for k in range(0, TPB, SCB):
            w = min(SCB, TPB - k)
            pltpu.sync_copy(acc_sh.at[pl.ds(base + k, w)],
                            rows_a.at[pl.ds(0, w)])
            pltpu.sync_copy(rows_a.at[pl.ds(0, w)],
                            out.at[c, phase, pl.ds(base + k, w)])

    zero_acc()
    plsc.subcore_barrier()

    def step1(ib, carry):
        pltpu.sync_copy(srcp.at[c, s, ib], src_v)
        pltpu.sync_copy(dstp.at[c, s, ib], dst_v)
        pend = pltpu.async_copy(x2.at[src_v.at[0]], rows[0], sems[0])
        scs = [None, None]
        for j in range(IB):
            pend.wait()
            sc = pltpu.async_copy(rows[j % 2], acc_sh.at[dst_v.at[j]],
                                  ssems[j % 2], add=True)
            if j + 1 < IB:
                if scs[(j + 1) % 2] is not None:
                    scs[(j + 1) % 2].wait()
                pend = pltpu.async_copy(x2.at[src_v.at[j + 1]],
                                        rows[(j + 1) % 2], sems[(j + 1) % 2])
            scs[j % 2] = sc
        scs[(IB - 1) % 2].wait()
        scs[IB % 2].wait()
        return carry

    lax.fori_loop(0, NIB, step1, 0)
    plsc.subcore_barrier()
    copy_out(0)
    pltpu.sync_copy(ones128, rows_b)
    plsc.subcore_barrier()


    def step2(ib, carry):
        pltpu.sync_copy(dstp.at[c, s, ib], dst_v)
        descs = [pltpu.async_copy(rows_b, acc_sh.at[dst_v.at[j]], ssem_a,
                                  add=True) for j in range(IB)]
        for d in descs:
            d.wait()
        return carry

    lax.fori_loop(0, NIB, step2, 0)
    plsc.subcore_barrier()
    copy_out(1)


@functools.cache
def _sc_agg():
    return pl.kernel(
        _sc_agg_body,
        out_type=jax.ShapeDtypeStruct((NUM_VIEW, 2, NP, FT), jnp.float32),
        mesh=plsc.VectorSubcoreMesh(core_axis_name="c", subcore_axis_name="s"),
        scratch_types=[
            pltpu.VMEM((IB, CHUNK), jnp.int32),
            pltpu.VMEM((IB, CHUNK), jnp.int32),
            pltpu.VMEM((CHUNK, FT), jnp.float32),
            pltpu.VMEM((CHUNK, FT), jnp.float32),
            pltpu.VMEM_SHARED((NP, FT), jnp.float32),
            pltpu.SemaphoreType.DMA,
            pltpu.SemaphoreType.DMA,
            pltpu.SemaphoreType.DMA,
            pltpu.SemaphoreType.DMA,
        ],
    )


def _dense_body(acc, wg, bg, ws, bs, wp, bp, wa, wb, bd1, wd2, bd2,
                wo, bo, com, priv, rec):
    f32 = jnp.float32
    dg = jnp.maximum(acc[0, 1, :, 0:1] - acc[0, 0, :, 0:1], 1.0)
    a = acc[0, 0] / dg
    h = jnp.maximum(jnp.dot(a, wg[0], preferred_element_type=f32) + bg[0], 0.0)
    cc = jnp.dot(h, ws[0], preferred_element_type=f32) + bs[0]
    pp = jnp.dot(h, wp[0], preferred_element_type=f32) + bp[0]
    d1 = jnp.maximum(jnp.dot(cc, wa[0], preferred_element_type=f32)
                     + jnp.dot(pp, wb[0], preferred_element_type=f32)
                     + bd1[0], 0.0)
    d2 = jnp.dot(d1, wd2[0], preferred_element_type=f32) + bd2[0]
    r = jnp.dot(jnp.maximum(d2, 0.0), wo[0], preferred_element_type=f32) + bo[0]
    com[0] = cc
    priv[0] = pp
    rec[0] = r


def _row_spec(d):
    return pl.BlockSpec((1, ROWB, d), lambda v, i: (v, i, 0))


def _w_spec(r, c):
    return pl.BlockSpec((1, r, c), lambda v, i: (v, 0, 0))


_dense = pl.pallas_call(
    _dense_body,
    grid=(NUM_VIEW, NROWB),
    in_specs=[
        pl.BlockSpec((1, 2, ROWB, FT), lambda v, i: (v, 0, i, 0)),
        _w_spec(FT, HID),
        _w_spec(1, HID),
        _w_spec(HID, CDIM),
        _w_spec(1, CDIM),
        _w_spec(HID, PDIM),
        _w_spec(1, PDIM),
        _w_spec(CDIM, HID),
        _w_spec(PDIM, HID),
        _w_spec(1, HID),
        _w_spec(HID, FT),
        _w_spec(1, FT),
        _w_spec(FT, FT),
        _w_spec(1, FT),
    ],
    out_specs=[_row_spec(CDIM), _row_spec(PDIM), _row_spec(FT)],
    out_shape=[
        jax.ShapeDtypeStruct((NUM_VIEW, N, CDIM), jnp.float32),
        jax.ShapeDtypeStruct((NUM_VIEW, N, PDIM), jnp.float32),
        jax.ShapeDtypeStruct((NUM_VIEW, N, FT), jnp.float32),
    ],
)


def kernel(x, adj, W_gcn, b_gcn, W_s, b_s, W_p, b_p, W_d1, b_d1, W_d2, b_d2,
           W_out, b_out):
    x2 = x.reshape(NUM_VIEW * N, FT)
    voff = (jnp.arange(NUM_VIEW, dtype=jnp.int32) * N)[:, None]
    pad = EPAD - E
    src = adj[:, 0, :] + voff
    dst = adj[:, 1, :]
    if pad:
        pad_src = jnp.broadcast_to((jnp.arange(pad, dtype=jnp.int32) * 31) % N,
                                   (NUM_VIEW, pad)) + voff
        pad_dst = jnp.broadcast_to(
            N + (jnp.arange(pad, dtype=jnp.int32) % (NP - N)), (NUM_VIEW, pad))
        src = jnp.concatenate([src, pad_src], axis=1)
        dst = jnp.concatenate([dst, pad_dst], axis=1)
    srcp = src.reshape(NUM_VIEW, NTILE, NIB, IB, CHUNK)
    dstp = dst.reshape(NUM_VIEW, NTILE, NIB, IB, CHUNK)
    z128 = jnp.zeros((CHUNK, FT), jnp.float32)
    ones128 = jnp.ones((CHUNK, FT), jnp.float32)

    out = _sc_agg()(x2, srcp, dstp, z128, ones128)

    com, priv, rec = _dense(
        out, W_gcn, b_gcn.reshape(NUM_VIEW, 1, HID),
        W_s, b_s.reshape(NUM_VIEW, 1, CDIM),
        W_p, b_p.reshape(NUM_VIEW, 1, PDIM),
        W_d1[:, :CDIM], W_d1[:, CDIM:], b_d1.reshape(NUM_VIEW, 1, HID),
        W_d2, b_d2.reshape(NUM_VIEW, 1, FT),
        W_out, b_out.reshape(NUM_VIEW, 1, FT))
    return (com, priv, rec)


# speedup vs baseline: 7.0429x; 1.0478x over previous
import functools

import jax
import jax.numpy as jnp
from jax import lax
from jax.experimental import pallas as pl
from jax.experimental.pallas import tpu as pltpu
from jax.experimental.pallas import tpu_sc as plsc

NUM_VIEW = 2
N = 10000
E = 320000
FT = 128
HID = 128
CDIM = 64
PDIM = 64

NTILE = 16
CHUNK = 125
IB = 8
NCH = 160
NIB = NCH // IB
EPT = NCH * CHUNK
EPAD = EPT * NTILE
TPB = 632
NP = TPB * NTILE
SCB = 96

ROWB = 400
NROWB = N // ROWB


def _sc_agg_body(x2, srcp, dstp, z128, ones128, out, src_v, dst_v, rows_a,
                 rows_b, acc_sh, sem_a, sem_b, ssem_a, ssem_b):
    c = lax.axis_index("c")
    s = lax.axis_index("s")
    base = s * TPB
    rows = (rows_a, rows_b)
    sems = (sem_a, sem_b)
    ssems = (ssem_a, ssem_b)

    def zero_acc():
        pltpu.sync_copy(z128, rows_a)
        for k in range(0, TPB, SCB):
            w = min(SCB, TPB - k)
            pltpu.sync_copy(rows_a.at[pl.ds(0, w)],
                            acc_sh.at[pl.ds(base + k, w)])

    def copy_out(phase):
        for k in range(0, TPB, SCB):
            w = min(SCB, TPB - k)
            pltpu.sync_copy(acc_sh.at[pl.ds(base + k, w)],
                            rows_a.at[pl.ds(0, w)])
            pltpu.sync_copy(rows_a.at[pl.ds(0, w)],
                            out.at[c, phase, pl.ds(base + k, w)])

    zero_acc()
    plsc.subcore_barrier()

    def step1(ib, carry):
        pltpu.sync_copy(srcp.at[c, s, ib], src_v)
        pltpu.sync_copy(dstp.at[c, s, ib], dst_v)
        pend = pltpu.async_copy(x2.at[src_v.at[0]], rows[0], sems[0])
        scs = [None, None]
        for j in range(IB):
            pend.wait()
            sc = pltpu.async_copy(rows[j % 2], acc_sh.at[dst_v.at[j]],
                                  ssems[j % 2], add=True)
            if j + 1 < IB:
                if scs[(j + 1) % 2] is not None:
                    scs[(j + 1) % 2].wait()
                pend = pltpu.async_copy(x2.at[src_v.at[j + 1]],
                                        rows[(j + 1) % 2], sems[(j + 1) % 2])
            scs[j % 2] = sc
        scs[(IB - 1) % 2].wait()
        scs[IB % 2].wait()
        return carry

    lax.fori_loop(0, NIB, step1, 0)
    plsc.subcore_barrier()
    copy_out(0)
    pltpu.sync_copy(ones128, rows_b)
    plsc.subcore_barrier()


    def step2(ib, carry):
        pltpu.sync_copy(dstp.at[c, s, ib], dst_v)
        descs = [pltpu.async_copy(rows_b, acc_sh.at[dst_v.at[j]], ssem_a,
                                  add=True) for j in range(IB)]
        for d in descs:
            d.wait()
        return carry

    lax.fori_loop(0, NIB, step2, 0)
    plsc.subcore_barrier()
    copy_out(1)


@functools.cache
def _sc_agg():
    return pl.kernel(
        _sc_agg_body,
        out_type=jax.ShapeDtypeStruct((NUM_VIEW, 2, NP, FT), jnp.float32),
        mesh=plsc.VectorSubcoreMesh(core_axis_name="c", subcore_axis_name="s"),
        scratch_types=[
            pltpu.VMEM((IB, CHUNK), jnp.int32),
            pltpu.VMEM((IB, CHUNK), jnp.int32),
            pltpu.VMEM((CHUNK, FT), jnp.float32),
            pltpu.VMEM((CHUNK, FT), jnp.float32),
            pltpu.VMEM_SHARED((NP, FT), jnp.float32),
            pltpu.SemaphoreType.DMA,
            pltpu.SemaphoreType.DMA,
            pltpu.SemaphoreType.DMA,
            pltpu.SemaphoreType.DMA,
        ],
    )


def _dense_body(acc, wg, bg, ws, bs, wp, bp, wa, wb, bd1, wd2, bd2,
                wo, bo, com, priv, rec):
    f32 = jnp.float32
    dg = jnp.maximum(acc[0, 1, :, 0:1] - acc[0, 0, :, 0:1], 1.0)
    a = acc[0, 0] / dg
    h = jnp.maximum(jnp.dot(a, wg[0], preferred_element_type=f32) + bg[0], 0.0)
    cc = jnp.dot(h, ws[0], preferred_element_type=f32) + bs[0]
    pp = jnp.dot(h, wp[0], preferred_element_type=f32) + bp[0]
    d1 = jnp.maximum(jnp.dot(cc, wa[0], preferred_element_type=f32)
                     + jnp.dot(pp, wb[0], preferred_element_type=f32)
                     + bd1[0], 0.0)
    d2 = jnp.dot(d1, wd2[0], preferred_element_type=f32) + bd2[0]
    r = jnp.dot(jnp.maximum(d2, 0.0), wo[0], preferred_element_type=f32) + bo[0]
    com[0] = cc
    priv[0] = pp
    rec[0] = r


def _row_spec(d):
    return pl.BlockSpec((1, ROWB, d), lambda v, i: (v, i, 0))


def _w_spec(r, c):
    return pl.BlockSpec((1, r, c), lambda v, i: (v, 0, 0))


_dense = pl.pallas_call(
    _dense_body,
    grid=(NUM_VIEW, NROWB),
    in_specs=[
        pl.BlockSpec((1, 2, ROWB, FT), lambda v, i: (v, 0, i, 0)),
        _w_spec(FT, HID),
        _w_spec(1, HID),
        _w_spec(HID, CDIM),
        _w_spec(1, CDIM),
        _w_spec(HID, PDIM),
        _w_spec(1, PDIM),
        _w_spec(CDIM, HID),
        _w_spec(PDIM, HID),
        _w_spec(1, HID),
        _w_spec(HID, FT),
        _w_spec(1, FT),
        _w_spec(FT, FT),
        _w_spec(1, FT),
    ],
    out_specs=[_row_spec(CDIM), _row_spec(PDIM), _row_spec(FT)],
    out_shape=[
        jax.ShapeDtypeStruct((NUM_VIEW, N, CDIM), jnp.float32),
        jax.ShapeDtypeStruct((NUM_VIEW, N, PDIM), jnp.float32),
        jax.ShapeDtypeStruct((NUM_VIEW, N, FT), jnp.float32),
    ],
)


def kernel(x, adj, W_gcn, b_gcn, W_s, b_s, W_p, b_p, W_d1, b_d1, W_d2, b_d2,
           W_out, b_out):
    x2 = x.reshape(NUM_VIEW * N, FT)
    voff = (jnp.arange(NUM_VIEW, dtype=jnp.int32) * N)[:, None]
    pad = EPAD - E
    src = adj[:, 0, :] + voff
    dst = adj[:, 1, :]
    if pad:
        pad_src = jnp.broadcast_to((jnp.arange(pad, dtype=jnp.int32) * 31) % N,
                                   (NUM_VIEW, pad)) + voff
        pad_dst = jnp.broadcast_to(
            N + (jnp.arange(pad, dtype=jnp.int32) % (NP - N)), (NUM_VIEW, pad))
        src = jnp.concatenate([src, pad_src], axis=1)
        dst = jnp.concatenate([dst, pad_dst], axis=1)
    srcp = src.reshape(NUM_VIEW, NTILE, NIB, IB, CHUNK)
    dstp = dst.reshape(NUM_VIEW, NTILE, NIB, IB, CHUNK)
    z128 = jnp.zeros((CHUNK, FT), jnp.float32)
    ones128 = jnp.ones((CHUNK, FT), jnp.float32)

    out = _sc_agg()(x2, srcp, dstp, z128, ones128)

    com, priv, rec = _dense(
        out, W_gcn, b_gcn.reshape(NUM_VIEW, 1, HID),
        W_s, b_s.reshape(NUM_VIEW, 1, CDIM),
        W_p, b_p.reshape(NUM_VIEW, 1, PDIM),
        W_d1[:, :CDIM], W_d1[:, CDIM:], b_d1.reshape(NUM_VIEW, 1, HID),
        W_d2, b_d2.reshape(NUM_VIEW, 1, FT),
        W_out, b_out.reshape(NUM_VIEW, 1, FT))
    return (com, priv, rec)
